# baseline (device time: 25879 ns/iter reference)
import jax
import jax.numpy as jnp
from jax import lax
from jax.experimental import pallas as pl
from jax.experimental.pallas import tpu as pltpu

N_DEV = 4
B = 512
D = 128
H = 256
B_PER = B // N_DEV
N_LAYERS = 3
N_CHUNK = 2
C = B // N_CHUNK
N_PH = 1 + N_LAYERS * N_CHUNK


def kernel(x, Win0, Wout0, Win1, Wout1, Win2, Wout2):
    def body(x_ref, win0_ref, wout0_ref, win1_ref, wout1_ref,
             win2_ref, wout2_ref, out_ref,
             xg_ref, win_s, wout_s, part_ref, in_sems,
             send_sems, recv_sems):
        my = lax.axis_index("i")
        pending = []

        in_cps = [pltpu.make_async_copy(x_ref, xg_ref.at[my], in_sems.at[0])]
        for i, (wr, vr) in enumerate([(win0_ref, wout0_ref),
                                      (win1_ref, wout1_ref),
                                      (win2_ref, wout2_ref)]):
            in_cps.append(pltpu.make_async_copy(
                wr, win_s.at[i], in_sems.at[1 + 2 * i]))
            in_cps.append(pltpu.make_async_copy(
                vr, wout_s.at[i], in_sems.at[2 + 2 * i]))
        for cp in in_cps:
            cp.start()

        barrier_sem = pltpu.get_barrier_semaphore()
        for o in range(1, N_DEV):
            peer = lax.rem(my + o, N_DEV)
            pl.semaphore_signal(barrier_sem, inc=1, device_id=(peer,),
                                device_id_type=pl.DeviceIdType.MESH)
        pl.semaphore_wait(barrier_sem, N_DEV - 1)

        def send(src, dst, ph, j, peer):
            rdma = pltpu.make_async_remote_copy(
                src_ref=src, dst_ref=dst,
                send_sem=send_sems.at[ph, j], recv_sem=recv_sems.at[ph, j],
                device_id=(peer,), device_id_type=pl.DeviceIdType.MESH)
            rdma.start()
            pending.append(rdma)

        def wait_recv(dst, ph, j):
            rdma = pltpu.make_async_remote_copy(
                src_ref=dst, dst_ref=dst,
                send_sem=send_sems.at[ph, j], recv_sem=recv_sems.at[ph, j],
                device_id=(my,), device_id_type=pl.DeviceIdType.MESH)
            rdma.wait_recv()

        in_cps[0].wait()
        for o in range(1, N_DEV):
            peer = lax.rem(my + o, N_DEV)
            send(xg_ref.at[my], xg_ref.at[my], 0, o - 1, peer)

        for o in range(1, N_DEV):
            wait_recv(xg_ref.at[lax.rem(my - o + N_DEV, N_DEV)], 0, o - 1)
        act0 = xg_ref[...].reshape(B, D)

        W = {}

        def f(a, k):
            if k not in W:
                in_cps[1 + 2 * k].wait()
                in_cps[2 + 2 * k].wait()
                W[k] = (win_s[k], wout_s[k])
            win, wout = W[k]
            h = jnp.maximum(
                lax.dot(a, win, preferred_element_type=jnp.float32), 0.0)
            return lax.dot(h.astype(jnp.bfloat16), wout,
                           preferred_element_type=jnp.float32)

        for k in range(N_LAYERS):
            for c in range(N_CHUNK):
                ph = 1 + N_CHUNK * k + c
                if k == 0:
                    a = act0[c * C:(c + 1) * C]
                else:
                    ph_prev = 1 + N_CHUNK * (k - 1) + c
                    for o in range(1, N_DEV):
                        wait_recv(part_ref.at[k - 1, c,
                                              lax.rem(my - o + N_DEV, N_DEV)],
                                  ph_prev, o - 1)
                    a = (part_ref[k - 1, c].astype(jnp.float32)
                         .sum(axis=0).astype(jnp.bfloat16))
                p = f(a, k)
                part_ref[k, c, my] = p.astype(jnp.bfloat16)
                for o in range(1, N_DEV):
                    peer = lax.rem(my + o, N_DEV)
                    send(part_ref.at[k, c, my], part_ref.at[k, c, my],
                         ph, o - 1, peer)

        for c in range(N_CHUNK):
            ph_last = 1 + N_CHUNK * (N_LAYERS - 1) + c
            for o in range(1, N_DEV):
                wait_recv(part_ref.at[N_LAYERS - 1, c,
                                      lax.rem(my - o + N_DEV, N_DEV)],
                          ph_last, o - 1)
            out_ref[c * C:(c + 1) * C, :] = (
                part_ref[N_LAYERS - 1, c].astype(jnp.float32)
                .sum(axis=0).astype(jnp.bfloat16))

        for rdma in pending:
            rdma.wait_send()

    args = [a.astype(jnp.bfloat16)
            for a in (x, Win0, Wout0, Win1, Wout1, Win2, Wout2)]
    return pl.pallas_call(
        body,
        out_shape=jax.ShapeDtypeStruct((B, D), jnp.bfloat16),
        in_specs=[pl.BlockSpec(memory_space=pl.ANY)] * 7,
        out_specs=pl.BlockSpec(memory_space=pltpu.VMEM),
        scratch_shapes=[
            pltpu.VMEM((N_DEV, B_PER, D), jnp.bfloat16),
            pltpu.VMEM((N_LAYERS, D, H), jnp.bfloat16),
            pltpu.VMEM((N_LAYERS, H, D), jnp.bfloat16),
            pltpu.VMEM((N_LAYERS, N_CHUNK, N_DEV, C, D), jnp.bfloat16),
            pltpu.SemaphoreType.DMA((7,)),
            pltpu.SemaphoreType.DMA((N_PH, N_DEV - 1)),
            pltpu.SemaphoreType.DMA((N_PH, N_DEV - 1)),
        ],
        compiler_params=pltpu.CompilerParams(collective_id=0),
    )(*args)


# device time: 25518 ns/iter; 1.0141x vs baseline; 1.0141x over previous
import jax
import jax.numpy as jnp
from jax import lax
from jax.experimental import pallas as pl
from jax.experimental.pallas import tpu as pltpu

N_DEV = 4
B = 512
D = 128
B_PER = B // N_DEV
N_LAYERS = 3
N_CHUNK = 2
C = B // N_CHUNK
N_PH = 1 + N_LAYERS * N_CHUNK


def kernel(x, Win0, Wout0, Win1, Wout1, Win2, Wout2):
    def body(x_ref, win0_ref, wout0_ref, win1_ref, wout1_ref,
             win2_ref, wout2_ref, out_ref,
             xg_ref, part_ref, send_sems, recv_sems):
        my = lax.axis_index("i")
        pending = []
        win_refs = [win0_ref, win1_ref, win2_ref]
        wout_refs = [wout0_ref, wout1_ref, wout2_ref]

        left = lax.rem(my + N_DEV - 1, N_DEV)
        right = lax.rem(my + 1, N_DEV)
        barrier_sem = pltpu.get_barrier_semaphore()
        for _ in range(2):
            for nbr in (left, right):
                pl.semaphore_signal(barrier_sem, inc=1, device_id=(nbr,),
                                    device_id_type=pl.DeviceIdType.MESH)
            pl.semaphore_wait(barrier_sem, 2)

        def send(src, dst, ph, j, peer):
            rdma = pltpu.make_async_remote_copy(
                src_ref=src, dst_ref=dst,
                send_sem=send_sems.at[ph, j], recv_sem=recv_sems.at[ph, j],
                device_id=(peer,), device_id_type=pl.DeviceIdType.MESH)
            rdma.start()
            pending.append(rdma)

        def wait_recv(dst, ph, j):
            rdma = pltpu.make_async_remote_copy(
                src_ref=dst, dst_ref=dst,
                send_sem=send_sems.at[ph, j], recv_sem=recv_sems.at[ph, j],
                device_id=(my,), device_id_type=pl.DeviceIdType.MESH)
            rdma.wait_recv()

        xg_ref[my] = x_ref[...]
        for o in range(1, N_DEV):
            peer = lax.rem(my + o, N_DEV)
            send(xg_ref.at[my], xg_ref.at[my], 0, o - 1, peer)

        W = {0: (win0_ref[...], wout0_ref[...])}

        for o in range(1, N_DEV):
            wait_recv(xg_ref.at[lax.rem(my - o + N_DEV, N_DEV)], 0, o - 1)
        act0 = xg_ref[...].reshape(B, D)

        def f(a, k):
            win, wout = W[k]
            h = jnp.maximum(
                lax.dot(a, win, preferred_element_type=jnp.float32), 0.0)
            return lax.dot(h.astype(jnp.bfloat16), wout,
                           preferred_element_type=jnp.float32)

        for k in range(N_LAYERS):
            for c in range(N_CHUNK):
                ph = 1 + N_CHUNK * k + c
                if k == 0:
                    a = act0[c * C:(c + 1) * C]
                else:
                    ph_prev = 1 + N_CHUNK * (k - 1) + c
                    for o in range(1, N_DEV):
                        wait_recv(part_ref.at[k - 1, c,
                                              lax.rem(my - o + N_DEV, N_DEV)],
                                  ph_prev, o - 1)
                    a = (part_ref[k - 1, c].astype(jnp.float32)
                         .sum(axis=0).astype(jnp.bfloat16))
                p = f(a, k)
                part_ref[k, c, my] = p.astype(jnp.bfloat16)
                for o in range(1, N_DEV):
                    peer = lax.rem(my + o, N_DEV)
                    send(part_ref.at[k, c, my], part_ref.at[k, c, my],
                         ph, o - 1, peer)
                if c == 0 and k + 1 < N_LAYERS:
                    W[k + 1] = (win_refs[k + 1][...], wout_refs[k + 1][...])

        for c in range(N_CHUNK):
            ph_last = 1 + N_CHUNK * (N_LAYERS - 1) + c
            for o in range(1, N_DEV):
                wait_recv(part_ref.at[N_LAYERS - 1, c,
                                      lax.rem(my - o + N_DEV, N_DEV)],
                          ph_last, o - 1)
            out_ref[c * C:(c + 1) * C, :] = (
                part_ref[N_LAYERS - 1, c].astype(jnp.float32)
                .sum(axis=0).astype(jnp.bfloat16))

        for rdma in pending:
            rdma.wait_send()

    args = [a.astype(jnp.bfloat16)
            for a in (x, Win0, Wout0, Win1, Wout1, Win2, Wout2)]
    return pl.pallas_call(
        body,
        out_shape=jax.ShapeDtypeStruct((B, D), jnp.bfloat16),
        in_specs=[pl.BlockSpec(memory_space=pltpu.VMEM)] * 7,
        out_specs=pl.BlockSpec(memory_space=pltpu.VMEM),
        scratch_shapes=[
            pltpu.VMEM((N_DEV, B_PER, D), jnp.bfloat16),
            pltpu.VMEM((N_LAYERS, N_CHUNK, N_DEV, C, D), jnp.bfloat16),
            pltpu.SemaphoreType.DMA((N_PH, N_DEV - 1)),
            pltpu.SemaphoreType.DMA((N_PH, N_DEV - 1)),
        ],
        compiler_params=pltpu.CompilerParams(collective_id=0),
    )(*args)


# device time: 24377 ns/iter; 1.0616x vs baseline; 1.0468x over previous
import jax
import jax.numpy as jnp
from jax import lax
from jax.experimental import pallas as pl
from jax.experimental.pallas import tpu as pltpu

N_DEV = 4
B = 512
D = 128
B_PER = B // N_DEV
N_LAYERS = 3
N_CHUNK = 4
C = B // N_CHUNK
N_PH = 1 + N_LAYERS * N_CHUNK


def kernel(x, Win0, Wout0, Win1, Wout1, Win2, Wout2):
    def body(x_ref, win0_ref, wout0_ref, win1_ref, wout1_ref,
             win2_ref, wout2_ref, out_ref,
             xg_ref, part_ref, send_sems, recv_sems):
        my = lax.axis_index("i")
        pending = []
        win_refs = [win0_ref, win1_ref, win2_ref]
        wout_refs = [wout0_ref, wout1_ref, wout2_ref]

        left = lax.rem(my + N_DEV - 1, N_DEV)
        right = lax.rem(my + 1, N_DEV)
        barrier_sem = pltpu.get_barrier_semaphore()
        for _ in range(2):
            for nbr in (left, right):
                pl.semaphore_signal(barrier_sem, inc=1, device_id=(nbr,),
                                    device_id_type=pl.DeviceIdType.MESH)
            pl.semaphore_wait(barrier_sem, 2)

        def send(src, dst, ph, j, peer):
            rdma = pltpu.make_async_remote_copy(
                src_ref=src, dst_ref=dst,
                send_sem=send_sems.at[ph, j], recv_sem=recv_sems.at[ph, j],
                device_id=(peer,), device_id_type=pl.DeviceIdType.MESH)
            rdma.start()
            pending.append(rdma)

        def wait_recv(dst, ph, j):
            rdma = pltpu.make_async_remote_copy(
                src_ref=dst, dst_ref=dst,
                send_sem=send_sems.at[ph, j], recv_sem=recv_sems.at[ph, j],
                device_id=(my,), device_id_type=pl.DeviceIdType.MESH)
            rdma.wait_recv()

        xg_ref[my] = x_ref[...]
        for o in range(1, N_DEV):
            peer = lax.rem(my + o, N_DEV)
            send(xg_ref.at[my], xg_ref.at[my], 0, o - 1, peer)

        W = {0: (win0_ref[...], wout0_ref[...])}

        for o in range(1, N_DEV):
            wait_recv(xg_ref.at[lax.rem(my - o + N_DEV, N_DEV)], 0, o - 1)
        act0 = xg_ref[...].reshape(B, D)

        def f(a, k):
            win, wout = W[k]
            h = jnp.maximum(
                lax.dot(a, win, preferred_element_type=jnp.float32), 0.0)
            return lax.dot(h.astype(jnp.bfloat16), wout,
                           preferred_element_type=jnp.float32)

        for k in range(N_LAYERS):
            for c in range(N_CHUNK):
                ph = 1 + N_CHUNK * k + c
                if k == 0:
                    a = act0[c * C:(c + 1) * C]
                else:
                    ph_prev = 1 + N_CHUNK * (k - 1) + c
                    for o in range(1, N_DEV):
                        wait_recv(part_ref.at[k - 1, c,
                                              lax.rem(my - o + N_DEV, N_DEV)],
                                  ph_prev, o - 1)
                    a = (part_ref[k - 1, c].astype(jnp.float32)
                         .sum(axis=0).astype(jnp.bfloat16))
                p = f(a, k)
                part_ref[k, c, my] = p.astype(jnp.bfloat16)
                for o in range(1, N_DEV):
                    peer = lax.rem(my + o, N_DEV)
                    send(part_ref.at[k, c, my], part_ref.at[k, c, my],
                         ph, o - 1, peer)
                if c == 0 and k + 1 < N_LAYERS:
                    W[k + 1] = (win_refs[k + 1][...], wout_refs[k + 1][...])

        for c in range(N_CHUNK):
            ph_last = 1 + N_CHUNK * (N_LAYERS - 1) + c
            for o in range(1, N_DEV):
                wait_recv(part_ref.at[N_LAYERS - 1, c,
                                      lax.rem(my - o + N_DEV, N_DEV)],
                          ph_last, o - 1)
            out_ref[c * C:(c + 1) * C, :] = (
                part_ref[N_LAYERS - 1, c].astype(jnp.float32)
                .sum(axis=0).astype(jnp.bfloat16))

        for rdma in pending:
            rdma.wait_send()

    args = [a.astype(jnp.bfloat16)
            for a in (x, Win0, Wout0, Win1, Wout1, Win2, Wout2)]
    return pl.pallas_call(
        body,
        out_shape=jax.ShapeDtypeStruct((B, D), jnp.bfloat16),
        in_specs=[pl.BlockSpec(memory_space=pltpu.VMEM)] * 7,
        out_specs=pl.BlockSpec(memory_space=pltpu.VMEM),
        scratch_shapes=[
            pltpu.VMEM((N_DEV, B_PER, D), jnp.bfloat16),
            pltpu.VMEM((N_LAYERS, N_CHUNK, N_DEV, C, D), jnp.bfloat16),
            pltpu.SemaphoreType.DMA((N_PH, N_DEV - 1)),
            pltpu.SemaphoreType.DMA((N_PH, N_DEV - 1)),
        ],
        compiler_params=pltpu.CompilerParams(collective_id=0),
    )(*args)


# device time: 24348 ns/iter; 1.0629x vs baseline; 1.0012x over previous
import jax
import jax.numpy as jnp
from jax import lax
from jax.experimental import pallas as pl
from jax.experimental.pallas import tpu as pltpu

N_DEV = 4
B = 512
D = 128
B_PER = B // N_DEV
N_LAYERS = 3
N_CHUNK = 4
C = B // N_CHUNK
N_PH = 1 + N_LAYERS * N_CHUNK


def kernel(x, Win0, Wout0, Win1, Wout1, Win2, Wout2):
    def body(x_ref, win0_ref, wout0_ref, win1_ref, wout1_ref,
             win2_ref, wout2_ref, out_ref,
             xg_ref, part_ref, send_sems, recv_sems):
        my = lax.axis_index("i")
        pending = []
        win_refs = [win0_ref, win1_ref, win2_ref]
        wout_refs = [wout0_ref, wout1_ref, wout2_ref]

        left = lax.rem(my + N_DEV - 1, N_DEV)
        right = lax.rem(my + 1, N_DEV)
        barrier_sem = pltpu.get_barrier_semaphore()
        for nbr in (left, right):
            pl.semaphore_signal(barrier_sem, inc=1, device_id=(nbr,),
                                device_id_type=pl.DeviceIdType.MESH)
        xg_ref[my] = x_ref[...]
        W = {0: (win0_ref[...], wout0_ref[...])}
        pl.semaphore_wait(barrier_sem, 2)
        for nbr in (left, right):
            pl.semaphore_signal(barrier_sem, inc=1, device_id=(nbr,),
                                device_id_type=pl.DeviceIdType.MESH)
        pl.semaphore_wait(barrier_sem, 2)

        def send(src, dst, ph, j, peer):
            rdma = pltpu.make_async_remote_copy(
                src_ref=src, dst_ref=dst,
                send_sem=send_sems.at[ph, j], recv_sem=recv_sems.at[ph, j],
                device_id=(peer,), device_id_type=pl.DeviceIdType.MESH)
            rdma.start()
            pending.append(rdma)

        def wait_recv(dst, ph, j):
            rdma = pltpu.make_async_remote_copy(
                src_ref=dst, dst_ref=dst,
                send_sem=send_sems.at[ph, j], recv_sem=recv_sems.at[ph, j],
                device_id=(my,), device_id_type=pl.DeviceIdType.MESH)
            rdma.wait_recv()

        for o in range(1, N_DEV):
            peer = lax.rem(my + o, N_DEV)
            send(xg_ref.at[my], xg_ref.at[my], 0, o - 1, peer)

        for o in range(1, N_DEV):
            wait_recv(xg_ref.at[lax.rem(my - o + N_DEV, N_DEV)], 0, o - 1)
        act0 = xg_ref[...].reshape(B, D)

        def f(a, k):
            win, wout = W[k]
            h = jnp.maximum(
                lax.dot(a, win, preferred_element_type=jnp.float32), 0.0)
            return lax.dot(h.astype(jnp.bfloat16), wout,
                           preferred_element_type=jnp.float32)

        for k in range(N_LAYERS):
            for c in range(N_CHUNK):
                ph = 1 + N_CHUNK * k + c
                if k == 0:
                    a = act0[c * C:(c + 1) * C]
                else:
                    ph_prev = 1 + N_CHUNK * (k - 1) + c
                    for o in range(1, N_DEV):
                        wait_recv(part_ref.at[k - 1, c,
                                              lax.rem(my - o + N_DEV, N_DEV)],
                                  ph_prev, o - 1)
                    a = (part_ref[k - 1, c].astype(jnp.float32)
                         .sum(axis=0).astype(jnp.bfloat16))
                p = f(a, k)
                part_ref[k, c, my] = p.astype(jnp.bfloat16)
                for o in range(1, N_DEV):
                    peer = lax.rem(my + o, N_DEV)
                    send(part_ref.at[k, c, my], part_ref.at[k, c, my],
                         ph, o - 1, peer)
                if c == 0 and k + 1 < N_LAYERS:
                    W[k + 1] = (win_refs[k + 1][...], wout_refs[k + 1][...])

        for c in range(N_CHUNK):
            ph_last = 1 + N_CHUNK * (N_LAYERS - 1) + c
            for o in range(1, N_DEV):
                wait_recv(part_ref.at[N_LAYERS - 1, c,
                                      lax.rem(my - o + N_DEV, N_DEV)],
                          ph_last, o - 1)
            out_ref[c * C:(c + 1) * C, :] = (
                part_ref[N_LAYERS - 1, c].astype(jnp.float32)
                .sum(axis=0).astype(jnp.bfloat16))

        for rdma in pending:
            rdma.wait_send()

    args = [a.astype(jnp.bfloat16)
            for a in (x, Win0, Wout0, Win1, Wout1, Win2, Wout2)]
    return pl.pallas_call(
        body,
        out_shape=jax.ShapeDtypeStruct((B, D), jnp.bfloat16),
        in_specs=[pl.BlockSpec(memory_space=pltpu.VMEM)] * 7,
        out_specs=pl.BlockSpec(memory_space=pltpu.VMEM),
        scratch_shapes=[
            pltpu.VMEM((N_DEV, B_PER, D), jnp.bfloat16),
            pltpu.VMEM((N_LAYERS, N_CHUNK, N_DEV, C, D), jnp.bfloat16),
            pltpu.SemaphoreType.DMA((N_PH, N_DEV - 1)),
            pltpu.SemaphoreType.DMA((N_PH, N_DEV - 1)),
        ],
        compiler_params=pltpu.CompilerParams(collective_id=0),
    )(*args)
